# baseline (speedup 1.0000x reference)
import functools

import jax
import jax.numpy as jnp
from jax import lax
from jax.experimental import pallas as pl
from jax.experimental.pallas import tpu as pltpu
from jax.experimental.pallas import tpu_sc as plsc

N = 10000
E = 320000
D = 128
NP = 10112
BLK = 2528
NTILES = 32
CH = 128
NB = 2
NBD = 4
EPT_BIG = 10240
G_BIG = EPT_BIG // (NB * CH)
G_SMALL = (E - 31 * EPT_BIG) // (NB * CH)
GD_BIG = EPT_BIG // (NBD * CH)
GD_SMALL = (E - 31 * EPT_BIG) // (NBD * CH)
SLAB = NP // 16


def _gelu(v):
    return 0.5 * v * (1.0 + lax.erf(v * 0.7071067811865476))


def _enc_body(x_ref, w0t, b0, w1t, b1, o_ref):
    h = _gelu(jnp.dot(x_ref[...], w0t[...], preferred_element_type=jnp.float32)
              + b0[...])
    o_ref[...] = (jnp.dot(h, w1t[...], preferred_element_type=jnp.float32)
                  + b1[...])


def _tc_encoder(x, w0t, b0, w1t, b1):
    return pl.pallas_call(
        _enc_body,
        grid=(NP // BLK,),
        in_specs=[
            pl.BlockSpec((BLK, D), lambda i: (i, 0)),
            pl.BlockSpec((D, D), lambda i: (0, 0)),
            pl.BlockSpec((D,), lambda i: (0,)),
            pl.BlockSpec((D, D), lambda i: (0, 0)),
            pl.BlockSpec((D,), lambda i: (0,)),
        ],
        out_specs=pl.BlockSpec((BLK, D), lambda i: (i, 0)),
        out_shape=jax.ShapeDtypeStruct((NP, D), jnp.float32),
    )(x, w0t, b0, w1t, b1)


def _combine_body(p_ref, c_ref, h_ref, wlt, bl, wrt, o_ref):
    agg = p_ref[0] + p_ref[1]
    den = jnp.maximum(c_ref[0, :, 0:1] + c_ref[1, :, 0:1], 1.0)
    o_ref[...] = _gelu(
        jnp.dot(agg / den, wlt[...], preferred_element_type=jnp.float32)
        + bl[...]
        + jnp.dot(h_ref[...], wrt[...], preferred_element_type=jnp.float32))


def _tc_combine(parts, cnt, h, wlt, bl, wrt):
    return pl.pallas_call(
        _combine_body,
        grid=(NP // BLK,),
        in_specs=[
            pl.BlockSpec((2, BLK, D), lambda i: (0, i, 0)),
            pl.BlockSpec((2, BLK, 8), lambda i: (0, i, 0)),
            pl.BlockSpec((BLK, D), lambda i: (i, 0)),
            pl.BlockSpec((D, D), lambda i: (0, 0)),
            pl.BlockSpec((D,), lambda i: (0,)),
            pl.BlockSpec((D, D), lambda i: (0, 0)),
        ],
        out_specs=pl.BlockSpec((BLK, D), lambda i: (i, 0)),
        out_shape=jax.ShapeDtypeStruct((NP, D), jnp.float32),
    )(parts, cnt, h, wlt, bl, wrt)


def _combdec_body(p_ref, c_ref, h_ref, wlt, bl, wrt,
                  w10t, b10, w11t, b11, w20t, b20, w21t, b21,
                  wa, wb, bc, o_ref):
    agg = p_ref[0] + p_ref[1]
    den = jnp.maximum(c_ref[0, :, 0:1] + c_ref[1, :, 0:1], 1.0)
    h = _gelu(
        jnp.dot(agg / den, wlt[...], preferred_element_type=jnp.float32)
        + bl[...]
        + jnp.dot(h_ref[...], wrt[...], preferred_element_type=jnp.float32))
    y1 = _gelu(jnp.dot(h, w10t[...], preferred_element_type=jnp.float32)
               + b10[...])
    y1 = _gelu(jnp.dot(y1, w11t[...], preferred_element_type=jnp.float32)
               + b11[...])
    y2 = _gelu(jnp.dot(h, w20t[...], preferred_element_type=jnp.float32)
               + b20[...])
    y2 = _gelu(jnp.dot(y2, w21t[...], preferred_element_type=jnp.float32)
               + b21[...])
    o_ref[...] = (jnp.dot(y1, wa[...], preferred_element_type=jnp.float32)
                  + jnp.dot(y2, wb[...], preferred_element_type=jnp.float32)
                  + bc[...])


def _tc_combine_dec(parts, cnt, h, wlt, bl, wrt,
                    w10t, b10, w11t, b11, w20t, b20, w21t, b21, wa, wb, bc):
    mat = pl.BlockSpec((D, D), lambda i: (0, 0))
    vec = pl.BlockSpec((D,), lambda i: (0,))
    return pl.pallas_call(
        _combdec_body,
        grid=(NP // BLK,),
        in_specs=[pl.BlockSpec((2, BLK, D), lambda i: (0, i, 0)),
                  pl.BlockSpec((2, BLK, 8), lambda i: (0, i, 0)),
                  pl.BlockSpec((BLK, D), lambda i: (i, 0)),
                  mat, vec, mat,
                  mat, vec, mat, vec, mat, vec, mat, vec, mat, mat, vec],
        out_specs=pl.BlockSpec((BLK, D), lambda i: (i, 0)),
        out_shape=jax.ShapeDtypeStruct((NP, D), jnp.float32),
    )(parts, cnt, h, wlt, bl, wrt,
      w10t, b10, w11t, b11, w20t, b20, w21t, b21, wa, wb, bc)


_MESH = plsc.VectorSubcoreMesh(core_axis_name="c", subcore_axis_name="s")


def _zero_block(buf):
    z = jnp.zeros((16,), jnp.float32)

    def body(r, _):
        for j in range(buf.shape[1] // 16):
            buf[r, pl.ds(j * 16, 16)] = z
        return 0

    lax.fori_loop(0, buf.shape[0], body, 0)


def _zero_slab(zsrc, acc_sh, s):
    base = s * SLAB
    pltpu.sync_copy(zsrc, acc_sh.at[pl.ds(base, CH)])
    pltpu.sync_copy(zsrc, acc_sh.at[pl.ds(base + CH, CH)])
    pltpu.sync_copy(zsrc, acc_sh.at[pl.ds(base + 2 * CH, CH)])
    pltpu.sync_copy(zsrc, acc_sh.at[pl.ds(base + 3 * CH, CH)])
    pltpu.sync_copy(zsrc.at[pl.ds(0, SLAB - 4 * CH)],
                    acc_sh.at[pl.ds(base + 4 * CH, SLAB - 4 * CH)])


@functools.partial(
    pl.kernel,
    out_type=jax.ShapeDtypeStruct((2, NP, D), jnp.float32),
    mesh=_MESH,
    scratch_types=[
        pltpu.VMEM((CH,), jnp.int32),
        pltpu.VMEM((CH,), jnp.int32),
        pltpu.VMEM((CH,), jnp.int32),
        pltpu.VMEM((CH,), jnp.int32),
        pltpu.VMEM((CH,), jnp.int32),
        pltpu.VMEM((CH,), jnp.int32),
        pltpu.VMEM((CH,), jnp.int32),
        pltpu.VMEM((CH,), jnp.int32),
        pltpu.VMEM((CH, D), jnp.float32),
        pltpu.VMEM((CH, D), jnp.float32),
        pltpu.VMEM((120, D), jnp.float32),
        pltpu.VMEM_SHARED((NP, D), jnp.float32),
        pltpu.SemaphoreType.DMA,
        pltpu.SemaphoreType.DMA,
        pltpu.SemaphoreType.DMA,
        pltpu.SemaphoreType.DMA,
        pltpu.SemaphoreType.DMA,
        pltpu.SemaphoreType.DMA,
        pltpu.SemaphoreType.DMA,
        pltpu.SemaphoreType.DMA,
    ],
)
def _sc_segsum(h_hbm, src_hbm, dst_hbm, cntdep_hbm, out_hbm,
               sv0, sv1, sv2, sv3, dv0, dv1, dv2, dv3,
               rows0, rows1, zsrc, acc_sh,
               g0, g1, s0, s1, i0, i1, i2, i3):
    srcv = (sv0, sv1, sv2, sv3)
    dstv = (dv0, dv1, dv2, dv3)
    rows = (rows0, rows1)
    gsem = (g0, g1)
    ssem = (s0, s1)
    isem = (i0, i1, i2, i3)
    c = lax.axis_index("c")
    s = lax.axis_index("s")
    t = c * 16 + s
    ebase = t * EPT_BIG
    sg = jnp.where(t == NTILES - 1, GD_SMALL, GD_BIG)


    def idx_load(k, j):
        off = ebase + k * CH
        pltpu.async_copy(src_hbm.at[pl.ds(off, CH)], srcv[j], isem[j])
        pltpu.async_copy(dst_hbm.at[pl.ds(off, CH)], dstv[j], isem[j])

    def iwait(j):
        pltpu.make_async_copy(src_hbm.at[pl.ds(0, CH)], srcv[j],
                              isem[j]).wait()
        pltpu.make_async_copy(dst_hbm.at[pl.ds(0, CH)], dstv[j],
                              isem[j]).wait()

    def gather(b, j):
        pltpu.async_copy(h_hbm.at[srcv[j]], rows[b], gsem[b])

    def gwait(b):
        pltpu.make_async_copy(h_hbm.at[srcv[0]], rows[b], gsem[b]).wait()

    def scatter(b, j):
        pltpu.async_copy(rows[b], acc_sh.at[dstv[j]], ssem[b], add=True)

    def swait(b):
        pltpu.make_async_copy(rows[b], acc_sh.at[dstv[0]], ssem[b]).wait()

    for j in range(4):
        idx_load(j, j)
    iwait(0)
    gather(0, 0)
    _zero_block(zsrc)
    base = s * SLAB
    for q in range(5):
        pltpu.sync_copy(zsrc, acc_sh.at[pl.ds(base + q * 120, 120)])
    pltpu.sync_copy(zsrc.at[pl.ds(0, SLAB - 600)],
                    acc_sh.at[pl.ds(base + 600, SLAB - 600)])
    plsc.subcore_barrier()

    iwait(1)
    gwait(0)
    gather(1, 1)
    scatter(0, 0)

    @pl.loop(0, sg - 1, init_carry=1)
    def _main(g, kk):
        for cc in range(4):
            j = (cc + 1) % 4
            b2 = cc % 2
            iwait((j + 1) % 4)
            gwait(b2 ^ 1)
            swait(b2)
            gather(b2, (j + 1) % 4)
            idx_load(kk + cc + 3, (j + 3) % 4)
            scatter(b2 ^ 1, j)
        return kk + 4

    for cc in range(3):
        jm = (cc + 1) % 4
        bm = (cc + 1) % 2
        if cc < 2:
            iwait((jm + 1) % 4)
        gwait(bm)
        swait(cc % 2)
        if cc < 2:
            gather(cc % 2, (jm + 1) % 4)
        scatter(bm, jm)
    swait(1)

    plsc.subcore_barrier()
    pltpu.sync_copy(acc_sh.at[pl.ds(s * SLAB, SLAB)],
                    out_hbm.at[c, pl.ds(s * SLAB, SLAB)])


@functools.partial(
    pl.kernel,
    out_type=jax.ShapeDtypeStruct((2, NP, D), jnp.float32),
    mesh=_MESH,
    scratch_types=[
        pltpu.VMEM((CH,), jnp.int32),
        pltpu.VMEM((CH,), jnp.int32),
        pltpu.VMEM((CH,), jnp.int32),
        pltpu.VMEM((CH,), jnp.int32),
        pltpu.VMEM((CH, D), jnp.float32),
        pltpu.VMEM_SHARED((NP, D), jnp.float32),
        pltpu.SemaphoreType.DMA,
        pltpu.SemaphoreType.DMA,
        pltpu.SemaphoreType.DMA,
        pltpu.SemaphoreType.DMA,
    ],
)
def _sc_degree(dst_hbm, out_hbm, dv0, dv1, dv2, dv3, ones_v, cnt_sh,
               s0, s1, s2, s3):
    dstv = (dv0, dv1, dv2, dv3)
    ssem = (s0, s1, s2, s3)
    c = lax.axis_index("c")
    s = lax.axis_index("s")
    t = c * 16 + s
    ebase = t * EPT_BIG
    gt = jnp.where(t == NTILES - 1, GD_SMALL, GD_BIG)

    _zero_block(ones_v)
    _zero_slab(ones_v, cnt_sh, s)

    one = jnp.ones((16,), jnp.float32)

    def fill(r, _):
        ones_v[r, pl.ds(0, 16)] = one
        return 0

    lax.fori_loop(0, CH, fill, 0)
    plsc.subcore_barrier()

    def scatter(k, b):
        off = ebase + k * CH
        pltpu.sync_copy(dst_hbm.at[pl.ds(off, CH)], dstv[b])
        pltpu.async_copy(ones_v, cnt_sh.at[dstv[b]], ssem[b], add=True)

    def swait(b):
        pltpu.make_async_copy(ones_v, cnt_sh.at[dstv[0]], ssem[b]).wait()

    for b in range(NBD):
        scatter(b, b)

    @pl.loop(1, gt)
    def _main(g):
        k0 = g * NBD
        for b in range(NBD):
            swait(b)
            scatter(k0 + b, b)

    for b in range(NBD):
        swait(b)

    plsc.subcore_barrier()
    pltpu.sync_copy(cnt_sh.at[pl.ds(s * SLAB, SLAB)],
                    out_hbm.at[c, pl.ds(s * SLAB, SLAB)])


def kernel(x, edge_index, enc_W0, enc_b0, enc_W1, enc_b1,
           conv0_Wl, conv0_bl, conv0_Wr,
           conv1_Wl, conv1_bl, conv1_Wr,
           conv2_Wl, conv2_bl, conv2_Wr,
           dec1_W0, dec1_b0, dec1_W1, dec1_b1, dec1_W2, dec1_b2,
           dec2_W0, dec2_b0, dec2_W1, dec2_b1, dec2_W2, dec2_b2):
    xp = jnp.pad(x, ((0, NP - N), (0, 0)))
    h = _tc_encoder(xp, enc_W0.T, enc_b0, enc_W1.T, enc_b1)

    src = edge_index[0]
    dst = edge_index[1]
    cnt = _sc_degree(dst)
    cnt8 = cnt[:, :, :8]

    wa = jnp.zeros((D, D), jnp.float32).at[:, 0:1].set(dec1_W2.T)
    wb = jnp.zeros((D, D), jnp.float32).at[:, 1:4].set(dec2_W2.T)
    bc = jnp.zeros((D,), jnp.float32).at[0].set(dec1_b2[0]).at[1:4].set(dec2_b2)

    for Wl, bl, Wr in ((conv0_Wl, conv0_bl, conv0_Wr),
                       (conv1_Wl, conv1_bl, conv1_Wr)):
        parts = _sc_segsum(h, src, dst, cnt)
        h = _tc_combine(parts, cnt8, h, Wl.T, bl, Wr.T)

    parts = _sc_segsum(h, src, dst, cnt)
    out = _tc_combine_dec(parts, cnt8, h, conv2_Wl.T, conv2_bl, conv2_Wr.T,
                          dec1_W0.T, dec1_b0, dec1_W1.T, dec1_b1,
                          dec2_W0.T, dec2_b0, dec2_W1.T, dec2_b1, wa, wb, bc)
    return out[:N, :4]


# speedup vs baseline: 1.0019x; 1.0019x over previous
import functools

import jax
import jax.numpy as jnp
from jax import lax
from jax.experimental import pallas as pl
from jax.experimental.pallas import tpu as pltpu
from jax.experimental.pallas import tpu_sc as plsc

N = 10000
E = 320000
D = 128
NP = 10112
BLK = 2528
NTILES = 32
CH = 128
NB = 2
NBD = 4
EPT_BIG = 10240
G_BIG = EPT_BIG // (NB * CH)
G_SMALL = (E - 31 * EPT_BIG) // (NB * CH)
GD_BIG = EPT_BIG // (NBD * CH)
GD_SMALL = (E - 31 * EPT_BIG) // (NBD * CH)
SLAB = NP // 16


def _gelu(v):
    return 0.5 * v * (1.0 + lax.erf(v * 0.7071067811865476))


def _enc_body(x_ref, w0t, b0, w1t, b1, o_ref):
    h = _gelu(jnp.dot(x_ref[...], w0t[...], preferred_element_type=jnp.float32)
              + b0[...])
    o_ref[...] = (jnp.dot(h, w1t[...], preferred_element_type=jnp.float32)
                  + b1[...])


def _tc_encoder(x, w0t, b0, w1t, b1):
    return pl.pallas_call(
        _enc_body,
        grid=(NP // BLK,),
        in_specs=[
            pl.BlockSpec((BLK, D), lambda i: (i, 0)),
            pl.BlockSpec((D, D), lambda i: (0, 0)),
            pl.BlockSpec((D,), lambda i: (0,)),
            pl.BlockSpec((D, D), lambda i: (0, 0)),
            pl.BlockSpec((D,), lambda i: (0,)),
        ],
        out_specs=pl.BlockSpec((BLK, D), lambda i: (i, 0)),
        out_shape=jax.ShapeDtypeStruct((NP, D), jnp.float32),
    )(x, w0t, b0, w1t, b1)


def _combine_body(p_ref, c_ref, h_ref, wlt, bl, wrt, o_ref):
    agg = p_ref[0] + p_ref[1]
    den = jnp.maximum(c_ref[0, :, 0:1] + c_ref[1, :, 0:1], 1.0)
    o_ref[...] = _gelu(
        jnp.dot(agg / den, wlt[...], preferred_element_type=jnp.float32)
        + bl[...]
        + jnp.dot(h_ref[...], wrt[...], preferred_element_type=jnp.float32))


def _tc_combine(parts, cnt, h, wlt, bl, wrt):
    return pl.pallas_call(
        _combine_body,
        grid=(NP // BLK,),
        in_specs=[
            pl.BlockSpec((2, BLK, D), lambda i: (0, i, 0)),
            pl.BlockSpec((2, BLK, 8), lambda i: (0, i, 0)),
            pl.BlockSpec((BLK, D), lambda i: (i, 0)),
            pl.BlockSpec((D, D), lambda i: (0, 0)),
            pl.BlockSpec((D,), lambda i: (0,)),
            pl.BlockSpec((D, D), lambda i: (0, 0)),
        ],
        out_specs=pl.BlockSpec((BLK, D), lambda i: (i, 0)),
        out_shape=jax.ShapeDtypeStruct((NP, D), jnp.float32),
    )(parts, cnt, h, wlt, bl, wrt)


def _combdec_body(p_ref, c_ref, h_ref, wlt, bl, wrt,
                  w10t, b10, w11t, b11, w20t, b20, w21t, b21,
                  wa, wb, bc, o_ref):
    agg = p_ref[0] + p_ref[1]
    den = jnp.maximum(c_ref[0, :, 0:1] + c_ref[1, :, 0:1], 1.0)
    h = _gelu(
        jnp.dot(agg / den, wlt[...], preferred_element_type=jnp.float32)
        + bl[...]
        + jnp.dot(h_ref[...], wrt[...], preferred_element_type=jnp.float32))
    y1 = _gelu(jnp.dot(h, w10t[...], preferred_element_type=jnp.float32)
               + b10[...])
    y1 = _gelu(jnp.dot(y1, w11t[...], preferred_element_type=jnp.float32)
               + b11[...])
    y2 = _gelu(jnp.dot(h, w20t[...], preferred_element_type=jnp.float32)
               + b20[...])
    y2 = _gelu(jnp.dot(y2, w21t[...], preferred_element_type=jnp.float32)
               + b21[...])
    o_ref[...] = (jnp.dot(y1, wa[...], preferred_element_type=jnp.float32)
                  + jnp.dot(y2, wb[...], preferred_element_type=jnp.float32)
                  + bc[...])


def _tc_combine_dec(parts, cnt, h, wlt, bl, wrt,
                    w10t, b10, w11t, b11, w20t, b20, w21t, b21, wa, wb, bc):
    mat = pl.BlockSpec((D, D), lambda i: (0, 0))
    vec = pl.BlockSpec((D,), lambda i: (0,))
    return pl.pallas_call(
        _combdec_body,
        grid=(NP // BLK,),
        in_specs=[pl.BlockSpec((2, BLK, D), lambda i: (0, i, 0)),
                  pl.BlockSpec((2, BLK, 8), lambda i: (0, i, 0)),
                  pl.BlockSpec((BLK, D), lambda i: (i, 0)),
                  mat, vec, mat,
                  mat, vec, mat, vec, mat, vec, mat, vec, mat, mat, vec],
        out_specs=pl.BlockSpec((BLK, D), lambda i: (i, 0)),
        out_shape=jax.ShapeDtypeStruct((NP, D), jnp.float32),
    )(parts, cnt, h, wlt, bl, wrt,
      w10t, b10, w11t, b11, w20t, b20, w21t, b21, wa, wb, bc)


_MESH = plsc.VectorSubcoreMesh(core_axis_name="c", subcore_axis_name="s")


def _zero_block(buf):
    z = jnp.zeros((16,), jnp.float32)

    def body(r, _):
        for j in range(buf.shape[1] // 16):
            buf[r, pl.ds(j * 16, 16)] = z
        return 0

    lax.fori_loop(0, buf.shape[0], body, 0)


def _zero_slab(zsrc, acc_sh, s):
    base = s * SLAB
    pltpu.sync_copy(zsrc, acc_sh.at[pl.ds(base, CH)])
    pltpu.sync_copy(zsrc, acc_sh.at[pl.ds(base + CH, CH)])
    pltpu.sync_copy(zsrc, acc_sh.at[pl.ds(base + 2 * CH, CH)])
    pltpu.sync_copy(zsrc, acc_sh.at[pl.ds(base + 3 * CH, CH)])
    pltpu.sync_copy(zsrc.at[pl.ds(0, SLAB - 4 * CH)],
                    acc_sh.at[pl.ds(base + 4 * CH, SLAB - 4 * CH)])


@functools.partial(
    pl.kernel,
    out_type=jax.ShapeDtypeStruct((2, NP, D), jnp.float32),
    mesh=_MESH,
    scratch_types=[
        pltpu.VMEM((CH,), jnp.int32),
        pltpu.VMEM((CH,), jnp.int32),
        pltpu.VMEM((CH,), jnp.int32),
        pltpu.VMEM((CH,), jnp.int32),
        pltpu.VMEM((CH,), jnp.int32),
        pltpu.VMEM((CH,), jnp.int32),
        pltpu.VMEM((CH,), jnp.int32),
        pltpu.VMEM((CH,), jnp.int32),
        pltpu.VMEM((CH, D), jnp.float32),
        pltpu.VMEM((CH, D), jnp.float32),
        pltpu.VMEM((120, D), jnp.float32),
        pltpu.VMEM_SHARED((NP, D), jnp.float32),
        pltpu.SemaphoreType.DMA,
        pltpu.SemaphoreType.DMA,
        pltpu.SemaphoreType.DMA,
        pltpu.SemaphoreType.DMA,
        pltpu.SemaphoreType.DMA,
        pltpu.SemaphoreType.DMA,
        pltpu.SemaphoreType.DMA,
        pltpu.SemaphoreType.DMA,
    ],
)
def _sc_segsum(h_hbm, src_hbm, dst_hbm, cntdep_hbm, out_hbm,
               sv0, sv1, sv2, sv3, dv0, dv1, dv2, dv3,
               rows0, rows1, zsrc, acc_sh,
               g0, g1, s0, s1, i0, i1, i2, i3):
    srcv = (sv0, sv1, sv2, sv3)
    dstv = (dv0, dv1, dv2, dv3)
    rows = (rows0, rows1)
    gsem = (g0, g1)
    ssem = (s0, s1)
    isem = (i0, i1, i2, i3)
    c = lax.axis_index("c")
    s = lax.axis_index("s")
    t = c * 16 + s
    ebase = t * EPT_BIG
    sg = jnp.where(t == NTILES - 1, GD_SMALL, GD_BIG)


    def idx_load(k, j):
        off = ebase + k * CH
        pltpu.async_copy(src_hbm.at[pl.ds(off, CH)], srcv[j], isem[j])
        pltpu.async_copy(dst_hbm.at[pl.ds(off, CH)], dstv[j], isem[j])

    def iwait(j):
        pltpu.make_async_copy(src_hbm.at[pl.ds(0, CH)], srcv[j],
                              isem[j]).wait()
        pltpu.make_async_copy(dst_hbm.at[pl.ds(0, CH)], dstv[j],
                              isem[j]).wait()

    def gather(b, j):
        pltpu.async_copy(h_hbm.at[srcv[j]], rows[b], gsem[b])

    def gwait(b):
        pltpu.make_async_copy(h_hbm.at[srcv[0]], rows[b], gsem[b]).wait()

    def scatter(b, j):
        pltpu.async_copy(rows[b], acc_sh.at[dstv[j]], ssem[b], add=True)

    def swait(b):
        pltpu.make_async_copy(rows[b], acc_sh.at[dstv[0]], ssem[b]).wait()

    for j in range(4):
        idx_load(j, j)
    iwait(0)
    gather(0, 0)
    _zero_block(zsrc)
    base = s * SLAB
    for q in range(5):
        pltpu.sync_copy(zsrc, acc_sh.at[pl.ds(base + q * 120, 120)])
    pltpu.sync_copy(zsrc.at[pl.ds(0, SLAB - 600)],
                    acc_sh.at[pl.ds(base + 600, SLAB - 600)])
    plsc.subcore_barrier()

    @pl.loop(0, sg - 1)
    def _main(g):
        k0 = g * 4
        for cc in range(4):
            b = cc % 2
            iwait((cc + 1) % 4)
            gwait(b)
            gather(b ^ 1, (cc + 1) % 4)
            scatter(b, cc)
            swait(b)
            idx_load(k0 + cc + 4, cc)

    for cc in range(4):
        b = cc % 2
        if cc < 3:
            iwait((cc + 1) % 4)
        gwait(b)
        if cc < 3:
            gather(b ^ 1, (cc + 1) % 4)
        scatter(b, cc)
        swait(b)

    plsc.subcore_barrier()
    pltpu.sync_copy(acc_sh.at[pl.ds(s * SLAB, SLAB)],
                    out_hbm.at[c, pl.ds(s * SLAB, SLAB)])


@functools.partial(
    pl.kernel,
    out_type=jax.ShapeDtypeStruct((2, NP, D), jnp.float32),
    mesh=_MESH,
    scratch_types=[
        pltpu.VMEM((CH,), jnp.int32),
        pltpu.VMEM((CH,), jnp.int32),
        pltpu.VMEM((CH,), jnp.int32),
        pltpu.VMEM((CH,), jnp.int32),
        pltpu.VMEM((CH, D), jnp.float32),
        pltpu.VMEM_SHARED((NP, D), jnp.float32),
        pltpu.SemaphoreType.DMA,
        pltpu.SemaphoreType.DMA,
        pltpu.SemaphoreType.DMA,
        pltpu.SemaphoreType.DMA,
    ],
)
def _sc_degree(dst_hbm, out_hbm, dv0, dv1, dv2, dv3, ones_v, cnt_sh,
               s0, s1, s2, s3):
    dstv = (dv0, dv1, dv2, dv3)
    ssem = (s0, s1, s2, s3)
    c = lax.axis_index("c")
    s = lax.axis_index("s")
    t = c * 16 + s
    ebase = t * EPT_BIG
    gt = jnp.where(t == NTILES - 1, GD_SMALL, GD_BIG)

    _zero_block(ones_v)
    _zero_slab(ones_v, cnt_sh, s)

    one = jnp.ones((16,), jnp.float32)

    def fill(r, _):
        ones_v[r, pl.ds(0, 16)] = one
        return 0

    lax.fori_loop(0, CH, fill, 0)
    plsc.subcore_barrier()

    def scatter(k, b):
        off = ebase + k * CH
        pltpu.sync_copy(dst_hbm.at[pl.ds(off, CH)], dstv[b])
        pltpu.async_copy(ones_v, cnt_sh.at[dstv[b]], ssem[b], add=True)

    def swait(b):
        pltpu.make_async_copy(ones_v, cnt_sh.at[dstv[0]], ssem[b]).wait()

    for b in range(NBD):
        scatter(b, b)

    @pl.loop(1, gt)
    def _main(g):
        k0 = g * NBD
        for b in range(NBD):
            swait(b)
            scatter(k0 + b, b)

    for b in range(NBD):
        swait(b)

    plsc.subcore_barrier()
    pltpu.sync_copy(cnt_sh.at[pl.ds(s * SLAB, SLAB)],
                    out_hbm.at[c, pl.ds(s * SLAB, SLAB)])


def kernel(x, edge_index, enc_W0, enc_b0, enc_W1, enc_b1,
           conv0_Wl, conv0_bl, conv0_Wr,
           conv1_Wl, conv1_bl, conv1_Wr,
           conv2_Wl, conv2_bl, conv2_Wr,
           dec1_W0, dec1_b0, dec1_W1, dec1_b1, dec1_W2, dec1_b2,
           dec2_W0, dec2_b0, dec2_W1, dec2_b1, dec2_W2, dec2_b2):
    xp = jnp.pad(x, ((0, NP - N), (0, 0)))
    h = _tc_encoder(xp, enc_W0.T, enc_b0, enc_W1.T, enc_b1)

    src = edge_index[0]
    dst = edge_index[1]
    cnt = _sc_degree(dst)
    cnt8 = cnt[:, :, :8]

    wa = jnp.zeros((D, D), jnp.float32).at[:, 0:1].set(dec1_W2.T)
    wb = jnp.zeros((D, D), jnp.float32).at[:, 1:4].set(dec2_W2.T)
    bc = jnp.zeros((D,), jnp.float32).at[0].set(dec1_b2[0]).at[1:4].set(dec2_b2)

    for Wl, bl, Wr in ((conv0_Wl, conv0_bl, conv0_Wr),
                       (conv1_Wl, conv1_bl, conv1_Wr)):
        parts = _sc_segsum(h, src, dst, cnt)
        h = _tc_combine(parts, cnt8, h, Wl.T, bl, Wr.T)

    parts = _sc_segsum(h, src, dst, cnt)
    out = _tc_combine_dec(parts, cnt8, h, conv2_Wl.T, conv2_bl, conv2_Wr.T,
                          dec1_W0.T, dec1_b0, dec1_W1.T, dec1_b1,
                          dec2_W0.T, dec2_b0, dec2_W1.T, dec2_b1, wa, wb, bc)
    return out[:N, :4]
